# baseline (speedup 1.0000x reference)
import functools

import jax
import jax.numpy as jnp
from jax import lax
from jax.experimental import pallas as pl
from jax.experimental.pallas import tpu as pltpu
from jax.experimental.pallas import tpu_sc as plsc

D_IN = 2048
D_SAE = 32768
K = 64
RBLK = 128
NRB = D_IN // RBLK
NW = 32
CW = D_IN // NW

_NEG = -3.0e38
_BIGI = 2**30


CH_R = 64
NCH = D_IN // CH_R
NBUF = 4


def _enc_body(x_ref, bdec_ref, w_hbm, benc_ref, vals_ref, idx_ref,
              wbuf, h_ref, sems):
    xc = x_ref[...] - bdec_ref[...]

    def copy(c, slot):
        return pltpu.make_async_copy(
            w_hbm.at[pl.ds(c * CH_R, CH_R), :], wbuf.at[slot], sems.at[slot])

    for c in range(NBUF):
        copy(c, c).start()
    for c in range(NCH):
        slot = c % NBUF
        copy(c, slot).wait()
        hb = jnp.dot(xc[:, c * CH_R:(c + 1) * CH_R], wbuf[slot],
                     preferred_element_type=jnp.float32)
        if c == 0:
            h_ref[...] = hb
        else:
            h_ref[...] = h_ref[...] + hb
        if c + NBUF < NCH:
            copy(c + NBUF, slot).start()

    ids = lax.broadcasted_iota(jnp.int32, (1, D_SAE), 1)
    k_iota = lax.broadcasted_iota(jnp.int32, (1, K), 1)

    def body(r, carry):
        h, vals, idxs = carry
        m = jnp.max(h)
        j = jnp.min(jnp.where(h == m, ids, _BIGI))
        h = jnp.where(ids == j, _NEG, h)
        vals = jnp.where(k_iota == r, m, vals)
        idxs = jnp.where(k_iota == r, j, idxs)
        return h, vals, idxs

    init = (jnp.maximum(h_ref[...] + benc_ref[...], 0.0),
            jnp.zeros((1, K), jnp.float32),
            jnp.zeros((1, K), jnp.int32))
    _, vals, idxs = lax.fori_loop(0, K, body, init)
    vals_ref[...] = vals
    idx_ref[...] = idxs


def _encode_topk(x, W_enc, b_enc, b_dec):
    return pl.pallas_call(
        _enc_body,
        in_specs=[
            pl.BlockSpec((1, D_IN), lambda: (0, 0)),
            pl.BlockSpec((1, D_IN), lambda: (0, 0)),
            pl.BlockSpec(memory_space=pl.ANY),
            pl.BlockSpec((1, D_SAE), lambda: (0, 0)),
        ],
        out_specs=[
            pl.BlockSpec((1, K), lambda: (0, 0)),
            pl.BlockSpec((1, K), lambda: (0, 0)),
        ],
        out_shape=[
            jax.ShapeDtypeStruct((1, K), jnp.float32),
            jax.ShapeDtypeStruct((1, K), jnp.int32),
        ],
        scratch_shapes=[
            pltpu.VMEM((NBUF, CH_R, D_SAE), jnp.float32),
            pltpu.VMEM((1, D_SAE), jnp.float32),
            pltpu.SemaphoreType.DMA((NBUF,)),
        ],
    )(x.reshape(1, D_IN), b_dec.reshape(1, D_IN), W_enc,
      b_enc.reshape(1, D_SAE))


def _sc_decode(w_flat, vals, ids, b_dec):
    mesh = plsc.VectorSubcoreMesh(core_axis_name="c", subcore_axis_name="s")

    @functools.partial(
        pl.kernel, mesh=mesh,
        out_type=jax.ShapeDtypeStruct((D_IN,), jnp.float32),
        scratch_types=[
            pltpu.VMEM((K,), jnp.int32),
            pltpu.VMEM((K,), jnp.float32),
            pltpu.VMEM((K, 128), jnp.float32),
            pltpu.VMEM((CW,), jnp.float32),
            pltpu.SemaphoreType.DMA,
        ],
    )
    def k(w_hbm, vals_hbm, ids_hbm, bdec_hbm, out_hbm,
          idx_v, vals_v, rows_v, acc_v, sem):
        wid = lax.axis_index("s") * 2 + lax.axis_index("c")
        pltpu.sync_copy(ids_hbm, idx_v)
        pltpu.sync_copy(vals_hbm, vals_v)
        blk = wid // 2
        pltpu.async_copy(w_hbm.at[idx_v, pl.ds(blk * 128, 128)],
                         rows_v, sem).wait()
        pltpu.sync_copy(bdec_hbm.at[pl.ds(wid * CW, CW)], acc_v)
        nl = CW // 16
        zero = jnp.zeros((16,), jnp.float32)
        acc_lo = [zero] * nl
        acc_hi = [zero] * nl
        for t in range(K // 16):
            vt = vals_v[pl.ds(t * 16, 16)]
            for i in range(16):
                val = vt[jnp.full((16,), i, jnp.int32)]
                row = rows_v.at[t * 16 + i]
                for l in range(nl):
                    acc_lo[l] = acc_lo[l] + row[pl.ds(l * 16, 16)] * val
                    acc_hi[l] = acc_hi[l] + row[pl.ds(CW + l * 16, 16)] * val
        hi_f = jnp.broadcast_to((wid % 2).astype(jnp.float32), (16,))
        for l in range(nl):
            blend = acc_lo[l] + hi_f * (acc_hi[l] - acc_lo[l])
            acc_v[pl.ds(l * 16, 16)] = acc_v[pl.ds(l * 16, 16)] + blend
        pltpu.sync_copy(acc_v, out_hbm.at[pl.ds(wid * CW, CW)])

    return k(w_flat, vals, ids, b_dec)


def kernel(x, W_enc, b_enc, W_dec, b_dec):
    vals, ids = _encode_topk(x, W_enc, b_enc, b_dec)
    out = _sc_decode(W_dec, vals.reshape(K), ids.reshape(K), b_dec)
    return out


# speedup vs baseline: 1.2745x; 1.2745x over previous
import functools

import jax
import jax.numpy as jnp
from jax import lax
from jax.experimental import pallas as pl
from jax.experimental.pallas import tpu as pltpu
from jax.experimental.pallas import tpu_sc as plsc

D_IN = 2048
D_SAE = 32768
K = 64
RBLK = 128
NRB = D_IN // RBLK
NW = 32
CW = D_IN // NW

_NEG = -3.0e38
_BIGI = 2**30


NS = 4
SB = RBLK // NS


def _enc_body(x_ref, bdec_ref, w0_ref, w1_ref, w2_ref, w3_ref, benc_ref,
              vals_ref, idx_ref, h_ref):
    i = pl.program_id(0)
    xc = x_ref[0] - bdec_ref[0]
    hb = jnp.dot(xc[:, 0 * SB:1 * SB], w0_ref[...],
                 preferred_element_type=jnp.float32)
    for q, wq in enumerate((w1_ref, w2_ref, w3_ref), start=1):
        hb = hb + jnp.dot(xc[:, q * SB:(q + 1) * SB], wq[...],
                          preferred_element_type=jnp.float32)

    @pl.when(i == 0)
    def _():
        h_ref[...] = hb

    @pl.when(i > 0)
    def _():
        h_ref[...] = h_ref[...] + hb

    @pl.when(i == NRB - 1)
    def _():
        hfull = jnp.maximum(h_ref[...] + benc_ref[...], 0.0)
        cw8 = D_SAE // 8
        hm = jnp.concatenate(
            [hfull[:, g * cw8:(g + 1) * cw8] for g in range(8)], axis=0)
        ids = (lax.broadcasted_iota(jnp.int32, (8, cw8), 0) * cw8
               + lax.broadcasted_iota(jnp.int32, (8, cw8), 1))
        k_iota = lax.broadcasted_iota(jnp.int32, (1, K), 1)

        def body(r, carry):
            h, vals, idxs = carry
            m = jnp.max(h)
            j = jnp.min(jnp.where(h == m, ids, _BIGI))
            h = jnp.where(ids == j, _NEG, h)
            vals = jnp.where(k_iota == r, m, vals)
            idxs = jnp.where(k_iota == r, j, idxs)
            return h, vals, idxs

        init = (hm,
                jnp.zeros((1, K), jnp.float32),
                jnp.zeros((1, K), jnp.int32))
        _, vals, idxs = lax.fori_loop(0, K, body, init)
        vals_ref[...] = vals
        idx_ref[...] = idxs


def _encode_topk(x, W_enc, b_enc, b_dec):
    return pl.pallas_call(
        _enc_body,
        grid=(NRB,),
        in_specs=[
            pl.BlockSpec((1, 1, RBLK), lambda i: (i, 0, 0)),
            pl.BlockSpec((1, 1, RBLK), lambda i: (i, 0, 0)),
            pl.BlockSpec((SB, D_SAE), lambda i: (NS * i + 0, 0)),
            pl.BlockSpec((SB, D_SAE), lambda i: (NS * i + 1, 0)),
            pl.BlockSpec((SB, D_SAE), lambda i: (NS * i + 2, 0)),
            pl.BlockSpec((SB, D_SAE), lambda i: (NS * i + 3, 0)),
            pl.BlockSpec((1, D_SAE), lambda i: (0, 0)),
        ],
        out_specs=[
            pl.BlockSpec((1, K), lambda i: (0, 0)),
            pl.BlockSpec((1, K), lambda i: (0, 0)),
        ],
        out_shape=[
            jax.ShapeDtypeStruct((1, K), jnp.float32),
            jax.ShapeDtypeStruct((1, K), jnp.int32),
        ],
        scratch_shapes=[pltpu.VMEM((1, D_SAE), jnp.float32)],
    )(x.reshape(NRB, 1, RBLK), b_dec.reshape(NRB, 1, RBLK),
      W_enc, W_enc, W_enc, W_enc, b_enc.reshape(1, D_SAE))


def _sc_decode(w_flat, vals, ids, b_dec):
    mesh = plsc.VectorSubcoreMesh(core_axis_name="c", subcore_axis_name="s")

    @functools.partial(
        pl.kernel, mesh=mesh,
        out_type=jax.ShapeDtypeStruct((D_IN,), jnp.float32),
        scratch_types=[
            pltpu.VMEM((K,), jnp.int32),
            pltpu.VMEM((K,), jnp.float32),
            pltpu.VMEM((K, 128), jnp.float32),
            pltpu.VMEM((CW,), jnp.float32),
            pltpu.SemaphoreType.DMA,
        ],
    )
    def k(w_hbm, vals_hbm, ids_hbm, bdec_hbm, out_hbm,
          idx_v, vals_v, rows_v, acc_v, sem):
        wid = lax.axis_index("s") * 2 + lax.axis_index("c")
        pltpu.sync_copy(ids_hbm, idx_v)
        pltpu.sync_copy(vals_hbm, vals_v)
        blk = wid // 2
        pltpu.async_copy(w_hbm.at[idx_v, pl.ds(blk * 128, 128)],
                         rows_v, sem).wait()
        pltpu.sync_copy(bdec_hbm.at[pl.ds(wid * CW, CW)], acc_v)
        nl = CW // 16
        zero = jnp.zeros((16,), jnp.float32)
        acc_lo = [zero] * nl
        acc_hi = [zero] * nl
        for t in range(K // 16):
            vt = vals_v[pl.ds(t * 16, 16)]
            for i in range(16):
                val = vt[jnp.full((16,), i, jnp.int32)]
                row = rows_v.at[t * 16 + i]
                for l in range(nl):
                    acc_lo[l] = acc_lo[l] + row[pl.ds(l * 16, 16)] * val
                    acc_hi[l] = acc_hi[l] + row[pl.ds(CW + l * 16, 16)] * val
        hi_f = jnp.broadcast_to((wid % 2).astype(jnp.float32), (16,))
        for l in range(nl):
            blend = acc_lo[l] + hi_f * (acc_hi[l] - acc_lo[l])
            acc_v[pl.ds(l * 16, 16)] = acc_v[pl.ds(l * 16, 16)] + blend
        pltpu.sync_copy(acc_v, out_hbm.at[pl.ds(wid * CW, CW)])

    return k(w_flat, vals, ids, b_dec)


def kernel(x, W_enc, b_enc, W_dec, b_dec):
    vals, ids = _encode_topk(x, W_enc, b_enc, b_dec)
    out = _sc_decode(W_dec, vals.reshape(K), ids.reshape(K), b_dec)
    return out
